# baseline (speedup 1.0000x reference)
import jax
import jax.numpy as jnp
from jax.experimental import pallas as pl
from jax.experimental.pallas import tpu as pltpu

D = 384
NEG_INF = float('-inf')


def _topk_mask(x, k):
    ax = x.ndim - 1

    def body(_, carry):
        mask, work = carry
        oh = work == jnp.max(work, axis=ax, keepdims=True)
        return (jnp.where(oh, 1.0, mask), jnp.where(oh, NEG_INF, work))

    mask, _ = jax.lax.fori_loop(0, k, body, (jnp.zeros_like(x), x))
    return mask


def _mlp_apply(x, w1_ref, c1_ref, a_ref, w2_ref, c2_ref):
    h = jnp.dot(x.astype(jnp.bfloat16), w1_ref[...],
                preferred_element_type=jnp.float32).astype(jnp.bfloat16)
    h = h + c1_ref[...]
    a = a_ref[0, 0].astype(jnp.bfloat16)
    h = jnp.where(h >= 0, h, a * h)
    return jnp.dot(h, w2_ref[...],
                   preferred_element_type=jnp.float32) + c2_ref[...]


def _enh_group(scr, get_ct, dist_scr, proto_scr):
    def dt(t, _):
        patt = scr[pl.ds(t * 5, 5)]
        ct = get_ct(t)
        diff = patt - ct[:, None, :]
        dist_scr[t] = jnp.sqrt(jnp.sum(diff * diff, axis=2))
        return 0

    jax.lax.fori_loop(0, 5, dt, 0)
    dist = dist_scr[...]
    d0 = dist[:, 0, :]
    other = jnp.sum(d0, axis=0, keepdims=True) - d0
    sim = dist / (other[:, None, :] + 1e-6)
    dist_scr[...] = _topk_mask(sim.reshape(25, 196), 30).reshape(5, 5, 196)

    def st(t, _):
        patt = scr[pl.ds(t * 5, 5)]
        mt = dist_scr[t]
        sel = jnp.sum(jnp.sum(mt[..., None] * patt, axis=1), axis=0,
                      keepdims=True) * (1.0 / 30.0)
        ct = get_ct(t)
        row = 2.0 * jnp.sum(ct, axis=0, keepdims=True) + sel
        proto_scr[pl.ds(t, 1)] = proto_scr[pl.ds(t, 1)] + row
        return 0

    jax.lax.fori_loop(0, 5, st, 0)


def _fused_kernel(des_ref, dpe_ref, eps_ref, epq_ref, ssv_ref, qsv_ref,
                  daw1, dac1, daa, daw2, dac2,
                  paw1, pac1, paa, paw2, pac2,
                  proto_ref, cls_ref,
                  d2_scr, eps2_scr, epq2_scr, dpe2_scr, proto_scr, na_scr,
                  num_scr, ws_scr, dist_scr):
    i = pl.program_id(0)
    da = (daw1, dac1, daa, daw2, dac2)
    pa = (paw1, pac1, paa, paw2, pac2)

    @pl.when(i == 0)
    def _():
        d2_scr[...] = _mlp_apply(des_ref[...].reshape(25, D), *da).reshape(5, 5, D)

    @pl.when(i < 25)
    def _():
        y_da = _mlp_apply(dpe_ref[...].reshape(196, D), *da)
        x2 = jnp.concatenate([eps_ref[...].reshape(196, D),
                              epq_ref[...].reshape(588, D), y_da], axis=0)
        y2 = x2 + _mlp_apply(x2, *pa)
        eps2_scr[pl.ds(i, 1)] = y2[:196].reshape(1, 196, D)
        epq2_scr[pl.ds(i * 3, 3)] = y2[196:784].reshape(3, 196, D)
        dpe2_scr[pl.ds(i, 1)] = y2[784:].reshape(1, 196, D)

    @pl.when(i == 25)
    def _():
        proto_scr[...] = jnp.zeros((5, D), jnp.float32)
        _enh_group(eps2_scr,
                   lambda t: ssv_ref[pl.ds(t * 5, 5)].reshape(5, D),
                   dist_scr, proto_scr)
        _enh_group(dpe2_scr,
                   lambda t: d2_scr[t],
                   dist_scr, proto_scr)
        proto = proto_scr[...] * 0.1
        proto_ref[...] = proto
        proto_scr[...] = proto

        def naj(j, _):
            pat = epq2_scr[pl.ds(j * 15, 15)]
            na = jnp.sqrt(jnp.sum(pat * pat, axis=2))
            na_scr[pl.ds(j * 15, 15)] = na[:, None, :]
            return 0

        jax.lax.fori_loop(0, 5, naj, 0)

    @pl.when(i == 26)
    def _():
        proto = proto_scr[...]
        nb = jnp.sqrt(jnp.sum(proto * proto, axis=1, keepdims=True))

        def numq(q, _):
            patq = epq2_scr[q]
            num_scr[q] = jax.lax.dot_general(
                proto, patq, (((1,), (1,)), ((), ())),
                precision=jax.lax.Precision.HIGHEST,
                preferred_element_type=jnp.float32)
            return 0

        jax.lax.fori_loop(0, 75, numq, 0)
        nbr = nb.reshape(1, 5, 1)

        def chunk(j, _):
            num = num_scr[pl.ds(j * 15, 15)]
            na = na_scr[pl.ds(j * 15, 15)]
            cos = num / jnp.maximum(na * nbr, 1e-8)
            m = jnp.max(cos, axis=2, keepdims=True)
            ex = jnp.exp(cos - m)
            w = ex / jnp.sum(ex, axis=2, keepdims=True)
            num_scr[pl.ds(j * 15, 15)] = _topk_mask(w, 30) * w
            return 0

        jax.lax.fori_loop(0, 5, chunk, 0)

        def wsq(q, _):
            ws_scr[q] = jax.lax.dot_general(
                num_scr[q], epq2_scr[q], (((1,), (0,)), ((), ())),
                preferred_element_type=jnp.float32).astype(jnp.bfloat16)
            return 0

        jax.lax.fori_loop(0, 75, wsq, 0)
        ws = ws_scr[...].astype(jnp.float32)
        q2 = 2.0 * qsv_ref[...].reshape(75, D)
        rows = [q2 + ws[:, e, :] for e in range(5)]
        cls_ref[...] = jnp.stack(rows, axis=0)


def kernel(support_set_vectors, query_set_vectors, dalle_emb_support,
           emb_patch_support, emb_patch_query, dalle_patch_embedding, params):
    p = params
    s = 1.0 / jnp.sqrt(jnp.float32(1.0 + 1e-5))

    def fold(pfx):
        g1s = p[pfx + 'bn1_g'] * s
        g2s = p[pfx + 'bn2_g'] * s
        w1 = (p[pfx + 'fc1_w'].T * g1s[None, :]).astype(jnp.bfloat16)
        c1 = (p[pfx + 'fc1_b'] * g1s + p[pfx + 'bn1_b']).reshape(1, D).astype(jnp.bfloat16)
        w2 = (p[pfx + 'fc2_w'].T * g2s[None, :]).astype(jnp.bfloat16)
        c2 = (p[pfx + 'fc2_b'] * g2s + p[pfx + 'bn2_b']).reshape(1, D)
        return (w1, c1, p[pfx + 'prelu'].reshape(1, 1), w2, c2)

    da_params = fold('da_')
    pa_params = fold('pa_')

    wspec = pl.BlockSpec((D, D), lambda i: (0, 0))
    cspec = pl.BlockSpec((1, D), lambda i: (0, 0))
    aspec = pl.BlockSpec((1, 1), lambda i: (0, 0))
    pspecs = [wspec, cspec, aspec, wspec, cspec]

    clamp = lambda i: (jnp.minimum(i, 24), 0, 0)
    proto, cls_ws = pl.pallas_call(
        _fused_kernel,
        grid=(27,),
        in_specs=[
            pl.BlockSpec((25, 1, D), lambda i: (0, 0, 0)),
            pl.BlockSpec((1, 196, D), clamp),
            pl.BlockSpec((1, 196, D), clamp),
            pl.BlockSpec((3, 196, D), clamp),
            pl.BlockSpec((25, 1, D), lambda i: (0, 0, 0)),
            pl.BlockSpec((75, 1, D), lambda i: (0, 0, 0)),
        ] + pspecs + pspecs,
        out_specs=[
            pl.BlockSpec((5, D), lambda i: (0, 0)),
            pl.BlockSpec((5, 75, D), lambda i: (0, 0, 0)),
        ],
        out_shape=[
            jax.ShapeDtypeStruct((5, D), jnp.float32),
            jax.ShapeDtypeStruct((5, 75, D), jnp.float32),
        ],
        scratch_shapes=[
            pltpu.VMEM((5, 5, D), jnp.float32),
            pltpu.VMEM((25, 196, D), jnp.float32),
            pltpu.VMEM((75, 196, D), jnp.float32),
            pltpu.VMEM((25, 196, D), jnp.float32),
            pltpu.VMEM((5, D), jnp.float32),
            pltpu.VMEM((75, 1, 196), jnp.float32),
            pltpu.VMEM((75, 5, 196), jnp.float32),
            pltpu.VMEM((75, 5, D), jnp.bfloat16),
            pltpu.VMEM((5, 5, 196), jnp.float32),
        ],
    )(dalle_emb_support, dalle_patch_embedding, emb_patch_support,
      emb_patch_query, support_set_vectors, query_set_vectors,
      *da_params, *pa_params)
    return (proto, cls_ws)
